# baseline (speedup 1.0000x reference)
import jax
import jax.numpy as jnp
from jax.experimental import pallas as pl
from jax.experimental.pallas import tpu as pltpu

_N = 65536
_NACT = 12
_EMB = 48
_D = (_NACT + 1) * _EMB


def _flow_body(x_ref, w_ref, b_ref, s_ref, o_ref):
    x = x_ref[...]
    y = jnp.dot(x, w_ref[...], preferred_element_type=jnp.float32)
    y = jnp.exp(y + b_ref[...])
    o_ref[...] = jnp.dot(y, s_ref[...], preferred_element_type=jnp.float32)


def _build_wbig(W, b):
    eye = jnp.eye(_NACT, dtype=W.dtype)
    top = jnp.concatenate([W, jnp.zeros((_EMB, _NACT), W.dtype)], axis=1)
    low = (W.T[:, :, None] * eye[:, None, :]).reshape(_NACT * _EMB, _NACT)
    low = jnp.concatenate([jnp.zeros((_NACT * _EMB, _NACT), W.dtype), low], axis=1)
    wbig = jnp.concatenate([top, low], axis=0)
    bbig = jnp.concatenate([b, b])[None, :]
    ones = jnp.ones((_NACT, 1), W.dtype)
    zs = jnp.zeros((_NACT, 1), W.dtype)
    sel = jnp.concatenate(
        [
            jnp.concatenate([zs, ones], axis=1),
            jnp.concatenate([ones, zs], axis=1),
        ],
        axis=0,
    )
    return wbig, bbig, sel


def kernel(inputs, W, b):
    x = inputs.astype(jnp.bfloat16).reshape(_N, _D)
    wbig, bbig, sel = _build_wbig(W, b)
    wbig = wbig.astype(jnp.bfloat16)
    bn = 4096
    grid = (_N // bn,)
    out = pl.pallas_call(
        _flow_body,
        grid=grid,
        in_specs=[
            pl.BlockSpec((bn, _D), lambda i: (i, 0)),
            pl.BlockSpec((_D, 2 * _NACT), lambda i: (0, 0)),
            pl.BlockSpec((1, 2 * _NACT), lambda i: (0, 0)),
            pl.BlockSpec((2 * _NACT, 2), lambda i: (0, 0)),
        ],
        out_specs=pl.BlockSpec((bn, 2), lambda i: (i, 0)),
        out_shape=jax.ShapeDtypeStruct((_N, 2), jnp.float32),
        compiler_params=pltpu.CompilerParams(
            dimension_semantics=("arbitrary",),
        ),
    )(x, wbig, bbig, sel)
    return out


# speedup vs baseline: 1.0562x; 1.0562x over previous
import jax
import jax.numpy as jnp
from jax.experimental import pallas as pl
from jax.experimental.pallas import tpu as pltpu

_N = 65536
_NACT = 12
_EMB = 48
_D = (_NACT + 1) * _EMB


def _flow_body(x_ref, w_ref, b_ref, s_ref, o_ref):
    x = x_ref[...]
    y = jnp.dot(x, w_ref[...], preferred_element_type=jnp.float32)
    y = jnp.exp(y + b_ref[...])
    o_ref[...] = jnp.dot(y, s_ref[...], preferred_element_type=jnp.float32)


def _build_wbig(W, b):
    eye = jnp.eye(_NACT, dtype=W.dtype)
    top = jnp.concatenate([W, jnp.zeros((_EMB, _NACT), W.dtype)], axis=1)
    low = (W.T[:, :, None] * eye[:, None, :]).reshape(_NACT * _EMB, _NACT)
    low = jnp.concatenate([jnp.zeros((_NACT * _EMB, _NACT), W.dtype), low], axis=1)
    wbig = jnp.concatenate([top, low], axis=0)
    bbig = jnp.concatenate([b, b])[None, :]
    ones = jnp.ones((_NACT, 1), W.dtype)
    zs = jnp.zeros((_NACT, 1), W.dtype)
    sel = jnp.concatenate(
        [
            jnp.concatenate([zs, ones], axis=1),
            jnp.concatenate([ones, zs], axis=1),
        ],
        axis=0,
    )
    return wbig, bbig, sel


def kernel(inputs, W, b):
    x = inputs.reshape(_N, _D)
    wbig, bbig, sel = _build_wbig(W, b)
    bn = 4096
    grid = (_N // bn,)
    out = pl.pallas_call(
        _flow_body,
        grid=grid,
        in_specs=[
            pl.BlockSpec((bn, _D), lambda i: (i, 0)),
            pl.BlockSpec((_D, 2 * _NACT), lambda i: (0, 0)),
            pl.BlockSpec((1, 2 * _NACT), lambda i: (0, 0)),
            pl.BlockSpec((2 * _NACT, 2), lambda i: (0, 0)),
        ],
        out_specs=pl.BlockSpec((bn, 2), lambda i: (i, 0)),
        out_shape=jax.ShapeDtypeStruct((_N, 2), jnp.float32),
        compiler_params=pltpu.CompilerParams(
            dimension_semantics=("arbitrary",),
        ),
    )(x, wbig, bbig, sel)
    return out
